# baseline (speedup 1.0000x reference)
import functools
import jax
import jax.numpy as jnp
from jax import lax
from jax.experimental import pallas as pl
from jax.experimental.pallas import tpu as pltpu
from jax.experimental.pallas import tpu_sc as plsc

_N = 8192
_D = 256
_K = 512
_MAX_ITER = 10
_ROWS = 512
_NC = 2
_NS = 16
_GPW = _K // (_NC * _NS)


def _sc_gather_body(x_hbm, idx_hbm, out_hbm, idx_v, rows_v, sem):
    wid = lax.axis_index("s") * _NC + lax.axis_index("c")
    base = wid * _GPW
    pltpu.sync_copy(idx_hbm.at[pl.ds(base, _GPW)], idx_v)
    pltpu.async_copy(x_hbm.at[idx_v], rows_v, sem).wait()
    pltpu.sync_copy(rows_v, out_hbm.at[pl.ds(base, _GPW)])


_sc_gather = functools.partial(
    pl.kernel,
    mesh=plsc.VectorSubcoreMesh(core_axis_name="c", subcore_axis_name="s"),
    out_type=jax.ShapeDtypeStruct((_K, _D), jnp.float32),
    scratch_types=[
        pltpu.VMEM((_GPW,), jnp.int32),
        pltpu.VMEM((_GPW, _D), jnp.float32),
        pltpu.SemaphoreType.DMA,
    ],
)(_sc_gather_body)


def _kmeans_body(x_ref, cinit_ref, out_ref, centers, sums, counts,
                 x2s, his, mids, los):
    it = pl.program_id(0)
    n_chunks = _N // _ROWS

    @pl.when(it == 0)
    def _init():
        centers[...] = cinit_ref[...]

        def prep(i, carry):
            xb = x_ref[pl.ds(i * _ROWS, _ROWS), :]
            x2 = jnp.sum(xb * xb, axis=1, keepdims=True)
            x2s[pl.ds(i * _ROWS, _ROWS), :] = jnp.broadcast_to(x2, (_ROWS, 8))
            hi = xb.astype(jnp.bfloat16)
            r1 = xb - hi.astype(jnp.float32)
            mid = r1.astype(jnp.bfloat16)
            lo = (r1 - mid.astype(jnp.float32)).astype(jnp.bfloat16)
            his[pl.ds(i * _ROWS, _ROWS), :] = hi
            mids[pl.ds(i * _ROWS, _ROWS), :] = mid
            los[pl.ds(i * _ROWS, _ROWS), :] = lo
            return carry

        jax.lax.fori_loop(0, n_chunks, prep, 0)

    sums[...] = jnp.zeros_like(sums)
    counts[...] = jnp.zeros_like(counts)

    c = centers[...]
    c2 = jnp.sum(c * c, axis=1)[None, :]
    iota = lax.broadcasted_iota(jnp.int32, (_ROWS, _K), 1).astype(jnp.float32)
    ones8 = jnp.ones((_ROWS, 8), jnp.bfloat16)
    dn = (((0,), (0,)), ((), ()))

    def body(i, carry):
        xb = x_ref[pl.ds(i * _ROWS, _ROWS), :]
        x2 = x2s[pl.ds(i * _ROWS, _ROWS), 0:1]
        xc = lax.dot_general(
            xb, c, (((1,), (1,)), ((), ())),
            precision=lax.Precision.DEFAULT,
            preferred_element_type=jnp.float32,
        )
        d2 = x2 + c2 - 2.0 * xc
        m = jnp.min(d2, axis=1, keepdims=True)
        masked = jnp.where(d2 <= m, iota, 4096.0)
        assign = jnp.min(masked, axis=1, keepdims=True)
        onehot = jnp.where(iota == assign, 1.0, 0.0)
        oh_bf = onehot.astype(jnp.bfloat16)
        sums[...] += (
            lax.dot_general(oh_bf, his[pl.ds(i * _ROWS, _ROWS), :], dn,
                            preferred_element_type=jnp.float32)
            + lax.dot_general(oh_bf, mids[pl.ds(i * _ROWS, _ROWS), :], dn,
                              preferred_element_type=jnp.float32)
            + lax.dot_general(oh_bf, los[pl.ds(i * _ROWS, _ROWS), :], dn,
                              preferred_element_type=jnp.float32)
        )
        counts[...] += lax.dot_general(
            oh_bf, ones8, dn,
            preferred_element_type=jnp.float32,
        )
        return carry

    for i in range(n_chunks):
        body(i, 0)

    cnt = jnp.maximum(counts[:, 0:1], 1.0)
    newc = sums[...] / cnt
    centers[...] = newc
    out_ref[...] = newc


@jax.jit
def kernel(x):
    perm = jax.random.permutation(jax.random.key(1), x.shape[0])[:_K]
    cinit = _sc_gather(x, perm.astype(jnp.int32))
    return pl.pallas_call(
        _kmeans_body,
        grid=(_MAX_ITER,),
        in_specs=[
            pl.BlockSpec((_N, _D), lambda i: (0, 0)),
            pl.BlockSpec((_K, _D), lambda i: (0, 0)),
        ],
        out_specs=pl.BlockSpec((_K, _D), lambda i: (0, 0)),
        out_shape=jax.ShapeDtypeStruct((_K, _D), jnp.float32),
        scratch_shapes=[
            pltpu.VMEM((_K, _D), jnp.float32),
            pltpu.VMEM((_K, _D), jnp.float32),
            pltpu.VMEM((_K, 8), jnp.float32),
            pltpu.VMEM((_N, 8), jnp.float32),
            pltpu.VMEM((_N, _D), jnp.bfloat16),
            pltpu.VMEM((_N, _D), jnp.bfloat16),
            pltpu.VMEM((_N, _D), jnp.bfloat16),
        ],
    )(x, cinit)


# speedup vs baseline: 1.0573x; 1.0573x over previous
import functools
import jax
import jax.numpy as jnp
from jax import lax
from jax.experimental import pallas as pl
from jax.experimental.pallas import tpu as pltpu
from jax.experimental.pallas import tpu_sc as plsc

_N = 8192
_D = 256
_K = 512
_MAX_ITER = 10
_ROWS = 2048
_NC = 2
_NS = 16
_GPW = _K // (_NC * _NS)


def _sc_gather_body(x_hbm, idx_hbm, out_hbm, idx_v, rows_v, sem):
    wid = lax.axis_index("s") * _NC + lax.axis_index("c")
    base = wid * _GPW
    pltpu.sync_copy(idx_hbm.at[pl.ds(base, _GPW)], idx_v)
    pltpu.async_copy(x_hbm.at[idx_v], rows_v, sem).wait()
    pltpu.sync_copy(rows_v, out_hbm.at[pl.ds(base, _GPW)])


_sc_gather = functools.partial(
    pl.kernel,
    mesh=plsc.VectorSubcoreMesh(core_axis_name="c", subcore_axis_name="s"),
    out_type=jax.ShapeDtypeStruct((_K, _D), jnp.float32),
    scratch_types=[
        pltpu.VMEM((_GPW,), jnp.int32),
        pltpu.VMEM((_GPW, _D), jnp.float32),
        pltpu.SemaphoreType.DMA,
    ],
)(_sc_gather_body)


def _kmeans_body(x_ref, cinit_ref, out_ref, centers, sums, counts,
                 x2s, his, mids, los):
    it = pl.program_id(0)
    n_chunks = _N // _ROWS

    @pl.when(it == 0)
    def _init():
        centers[...] = cinit_ref[...]

        def prep(i, carry):
            xb = x_ref[pl.ds(i * _ROWS, _ROWS), :]
            x2 = jnp.sum(xb * xb, axis=1, keepdims=True)
            x2s[pl.ds(i * _ROWS, _ROWS), :] = jnp.broadcast_to(x2, (_ROWS, 8))
            hi = xb.astype(jnp.bfloat16)
            r1 = xb - hi.astype(jnp.float32)
            mid = r1.astype(jnp.bfloat16)
            lo = (r1 - mid.astype(jnp.float32)).astype(jnp.bfloat16)
            his[pl.ds(i * _ROWS, _ROWS), :] = hi
            mids[pl.ds(i * _ROWS, _ROWS), :] = mid
            los[pl.ds(i * _ROWS, _ROWS), :] = lo
            return carry

        jax.lax.fori_loop(0, n_chunks, prep, 0)

    sums[...] = jnp.zeros_like(sums)
    counts[...] = jnp.zeros_like(counts)

    c = centers[...]
    c2 = jnp.sum(c * c, axis=1)[None, :]
    iota = lax.broadcasted_iota(jnp.int32, (_ROWS, _K), 1).astype(jnp.float32)
    ones8 = jnp.ones((_ROWS, 8), jnp.bfloat16)
    dn = (((0,), (0,)), ((), ()))

    def body(i, carry):
        xb = x_ref[pl.ds(i * _ROWS, _ROWS), :]
        x2 = x2s[pl.ds(i * _ROWS, _ROWS), 0:1]
        xc = lax.dot_general(
            xb, c, (((1,), (1,)), ((), ())),
            precision=lax.Precision.DEFAULT,
            preferred_element_type=jnp.float32,
        )
        d2 = x2 + c2 - 2.0 * xc
        m = jnp.min(d2, axis=1, keepdims=True)
        masked = jnp.where(d2 <= m, iota, 4096.0)
        assign = jnp.min(masked, axis=1, keepdims=True)
        onehot = jnp.where(iota == assign, 1.0, 0.0)
        oh_bf = onehot.astype(jnp.bfloat16)
        sums[...] += (
            lax.dot_general(oh_bf, his[pl.ds(i * _ROWS, _ROWS), :], dn,
                            preferred_element_type=jnp.float32)
            + lax.dot_general(oh_bf, mids[pl.ds(i * _ROWS, _ROWS), :], dn,
                              preferred_element_type=jnp.float32)
            + lax.dot_general(oh_bf, los[pl.ds(i * _ROWS, _ROWS), :], dn,
                              preferred_element_type=jnp.float32)
        )
        counts[...] += lax.dot_general(
            oh_bf, ones8, dn,
            preferred_element_type=jnp.float32,
        )
        return carry

    for i in range(n_chunks):
        body(i, 0)

    cnt = jnp.maximum(counts[:, 0:1], 1.0)
    newc = sums[...] / cnt
    centers[...] = newc
    out_ref[...] = newc


@jax.jit
def kernel(x):
    perm = jax.random.permutation(jax.random.key(1), x.shape[0])[:_K]
    cinit = _sc_gather(x, perm.astype(jnp.int32))
    return pl.pallas_call(
        _kmeans_body,
        grid=(_MAX_ITER,),
        in_specs=[
            pl.BlockSpec((_N, _D), lambda i: (0, 0)),
            pl.BlockSpec((_K, _D), lambda i: (0, 0)),
        ],
        out_specs=pl.BlockSpec((_K, _D), lambda i: (0, 0)),
        out_shape=jax.ShapeDtypeStruct((_K, _D), jnp.float32),
        scratch_shapes=[
            pltpu.VMEM((_K, _D), jnp.float32),
            pltpu.VMEM((_K, _D), jnp.float32),
            pltpu.VMEM((_K, 8), jnp.float32),
            pltpu.VMEM((_N, 8), jnp.float32),
            pltpu.VMEM((_N, _D), jnp.bfloat16),
            pltpu.VMEM((_N, _D), jnp.bfloat16),
            pltpu.VMEM((_N, _D), jnp.bfloat16),
        ],
    )(x, cinit)


# speedup vs baseline: 1.0592x; 1.0018x over previous
import functools
import jax
import jax.numpy as jnp
from jax import lax
from jax.experimental import pallas as pl
from jax.experimental.pallas import tpu as pltpu
from jax.experimental.pallas import tpu_sc as plsc

_N = 8192
_D = 256
_K = 512
_MAX_ITER = 10
_ROWS = 1024
_NC = 2
_NS = 16
_GPW = _K // (_NC * _NS)


def _sc_gather_body(x_hbm, idx_hbm, out_hbm, idx_v, rows_v, sem):
    wid = lax.axis_index("s") * _NC + lax.axis_index("c")
    base = wid * _GPW
    pltpu.sync_copy(idx_hbm.at[pl.ds(base, _GPW)], idx_v)
    pltpu.async_copy(x_hbm.at[idx_v], rows_v, sem).wait()
    pltpu.sync_copy(rows_v, out_hbm.at[pl.ds(base, _GPW)])


_sc_gather = functools.partial(
    pl.kernel,
    mesh=plsc.VectorSubcoreMesh(core_axis_name="c", subcore_axis_name="s"),
    out_type=jax.ShapeDtypeStruct((_K, _D), jnp.float32),
    scratch_types=[
        pltpu.VMEM((_GPW,), jnp.int32),
        pltpu.VMEM((_GPW, _D), jnp.float32),
        pltpu.SemaphoreType.DMA,
    ],
)(_sc_gather_body)


def _kmeans_body(x_ref, cinit_ref, out_ref, centers, sums, counts,
                 x2s, his, mids, los):
    it = pl.program_id(0)
    n_chunks = _N // _ROWS

    @pl.when(it == 0)
    def _init():
        centers[...] = cinit_ref[...]

        def prep(i, carry):
            xb = x_ref[pl.ds(i * _ROWS, _ROWS), :]
            x2 = jnp.sum(xb * xb, axis=1, keepdims=True)
            x2s[pl.ds(i * _ROWS, _ROWS), :] = jnp.broadcast_to(x2, (_ROWS, 8))
            hi = xb.astype(jnp.bfloat16)
            r1 = xb - hi.astype(jnp.float32)
            mid = r1.astype(jnp.bfloat16)
            lo = (r1 - mid.astype(jnp.float32)).astype(jnp.bfloat16)
            his[pl.ds(i * _ROWS, _ROWS), :] = hi
            mids[pl.ds(i * _ROWS, _ROWS), :] = mid
            los[pl.ds(i * _ROWS, _ROWS), :] = lo
            return carry

        jax.lax.fori_loop(0, n_chunks, prep, 0)

    sums[...] = jnp.zeros_like(sums)
    counts[...] = jnp.zeros_like(counts)

    c = centers[...]
    c2 = jnp.sum(c * c, axis=1)[None, :]
    iota = lax.broadcasted_iota(jnp.int32, (_ROWS, _K), 1).astype(jnp.float32)
    ones8 = jnp.ones((_ROWS, 8), jnp.bfloat16)
    dn = (((0,), (0,)), ((), ()))

    def body(i, carry):
        xb = x_ref[pl.ds(i * _ROWS, _ROWS), :]
        x2 = x2s[pl.ds(i * _ROWS, _ROWS), 0:1]
        xc = lax.dot_general(
            xb, c, (((1,), (1,)), ((), ())),
            precision=lax.Precision.DEFAULT,
            preferred_element_type=jnp.float32,
        )
        d2 = x2 + c2 - 2.0 * xc
        m = jnp.min(d2, axis=1, keepdims=True)
        masked = jnp.where(d2 <= m, iota, 4096.0)
        assign = jnp.min(masked, axis=1, keepdims=True)
        onehot = jnp.where(iota == assign, 1.0, 0.0)
        oh_bf = onehot.astype(jnp.bfloat16)
        sums[...] += (
            lax.dot_general(oh_bf, his[pl.ds(i * _ROWS, _ROWS), :], dn,
                            preferred_element_type=jnp.float32)
            + lax.dot_general(oh_bf, mids[pl.ds(i * _ROWS, _ROWS), :], dn,
                              preferred_element_type=jnp.float32)
            + lax.dot_general(oh_bf, los[pl.ds(i * _ROWS, _ROWS), :], dn,
                              preferred_element_type=jnp.float32)
        )
        counts[...] += lax.dot_general(
            oh_bf, ones8, dn,
            preferred_element_type=jnp.float32,
        )
        return carry

    for i in range(n_chunks):
        body(i, 0)

    cnt = jnp.maximum(counts[:, 0:1], 1.0)
    newc = sums[...] / cnt
    centers[...] = newc
    out_ref[...] = newc


@jax.jit
def kernel(x):
    perm = jax.random.permutation(jax.random.key(1), x.shape[0])[:_K]
    cinit = _sc_gather(x, perm.astype(jnp.int32))
    return pl.pallas_call(
        _kmeans_body,
        grid=(_MAX_ITER,),
        in_specs=[
            pl.BlockSpec((_N, _D), lambda i: (0, 0)),
            pl.BlockSpec((_K, _D), lambda i: (0, 0)),
        ],
        out_specs=pl.BlockSpec((_K, _D), lambda i: (0, 0)),
        out_shape=jax.ShapeDtypeStruct((_K, _D), jnp.float32),
        scratch_shapes=[
            pltpu.VMEM((_K, _D), jnp.float32),
            pltpu.VMEM((_K, _D), jnp.float32),
            pltpu.VMEM((_K, 8), jnp.float32),
            pltpu.VMEM((_N, 8), jnp.float32),
            pltpu.VMEM((_N, _D), jnp.bfloat16),
            pltpu.VMEM((_N, _D), jnp.bfloat16),
            pltpu.VMEM((_N, _D), jnp.bfloat16),
        ],
    )(x, cinit)
